# baseline (speedup 1.0000x reference)
import functools

import jax
import jax.numpy as jnp
from jax import lax
from jax.experimental import pallas as pl
from jax.experimental.pallas import tpu as pltpu
from jax.experimental.pallas import tpu_sc as plsc

B = 1024
NUM_AGENTS = 4
MAX_OBS = 60
NNF = 8
HID = 16
HEADS = 8
OUT_CH = 2
NPG = NUM_AGENTS + MAX_OBS
HF = HEADS * HID
NFL = NPG * NNF
N = B * NPG

GBT = 64
NWORK = 32
GPW = B // NWORK

f32 = jnp.float32
i32 = jnp.int32
HIGH = jax.lax.Precision.HIGHEST
DEF = jax.lax.Precision.DEFAULT


def _dg(a, b, dims, prec):
    return jax.lax.dot_general(a, b, dims, preferred_element_type=f32,
                               precision=prec)


def _mm(a, b, prec=HIGH):
    return _dg(a, b, (((a.ndim - 1,), (0,)), ((), ())), prec)


def _tc1_body(gfn_ref, srcf_ref, dstf_ref, ea_ref, W1_ref, As_ref, Ad_ref,
              xp_ref, sc_ref, eidx_ref, ef_ref):
    srcf = srcf_ref[...]
    dstf = dstf_ref[...]
    ea = ea_ref[...]
    xp = _dg(gfn_ref[...], W1_ref[...], (((1,), (0,)), ((), ())), DEF)
    xp_ref[...] = xp.reshape(GBT * NPG, HF)
    s_src = _mm(xp, As_ref[...])
    s_dst = _mm(xp, Ad_ref[...])
    sc_ref[...] = jnp.concatenate(
        [jnp.swapaxes(s_src, 1, 2), jnp.swapaxes(s_dst, 1, 2)], axis=2)
    z4 = jnp.zeros((GBT, NPG - MAX_OBS), i32)
    eidx_ref[...] = jnp.concatenate(
        [srcf.astype(i32), z4, dstf.astype(i32), z4], axis=1)
    iota = jax.lax.broadcasted_iota(i32, (GBT, MAX_OBS, NPG), 2)
    DstOH = (dstf.astype(i32)[:, :, None] == iota).astype(f32)
    ea_sum = _dg(ea, DstOH, (((1,), (1,)), ((0,), (0,))), HIGH)
    cnt = jnp.sum(DstOH, axis=1)
    loop_attr = ea_sum / jnp.maximum(cnt, 1.0)
    ef_ref[...] = jnp.concatenate(
        [ea, jnp.zeros((GBT, NPG - MAX_OBS), f32), loop_attr], axis=1)


def _tc2_body(agg_ref, b_ref, W_ref, As_ref, Ad_ref, xp_ref, sc_ref):
    x = jax.nn.relu(agg_ref[...] + b_ref[...])
    xp2 = _mm(x, W_ref[...], DEF)
    xp_ref[...] = xp2
    xp3 = xp2.reshape(GBT, NPG, HF)
    s_src = _mm(xp3, As_ref[...])
    s_dst = _mm(xp3, Ad_ref[...])
    sc_ref[...] = jnp.concatenate(
        [jnp.swapaxes(s_src, 1, 2), jnp.swapaxes(s_dst, 1, 2)], axis=2)


def _tc4_body(agg_ref, b_ref, fc1w_ref, fc1b_ref, fc2w_ref, fc2b_ref, o_ref):
    x = jax.nn.relu(agg_ref[...] + b_ref[...])
    x3 = x.reshape(GBT, NPG, HF)
    gemb = jnp.mean(x3, axis=1)
    aemb = x3[:, :NUM_AGENTS, :]
    gtile = jnp.broadcast_to(gemb[:, None, :], (GBT, NUM_AGENTS, HF))
    comb = jnp.concatenate([aemb, gtile], axis=2).reshape(
        GBT * NUM_AGENTS, 2 * HF)
    h = jax.nn.relu(_mm(comb, fc1w_ref[...], DEF) + fc1b_ref[...])
    o_ref[...] = _mm(h, fc2w_ref[...], DEF) + fc2b_ref[...]


def _sc_attn_body(add_loops, xp_hbm, sc_hbm, eidx_hbm, ef_hbm, c_hbm,
                  out_hbm, xp_v, out_v, sc_v, idx_v, ef_v, e_v, el_v,
                  den_v, c_v, w_v):
    wid = lax.axis_index("s") * 2 + lax.axis_index("c")
    lane = lax.iota(i32, 16)
    mask8 = lane < 8
    pltpu.sync_copy(c_hbm, c_v)

    def bcast(v):
        return jnp.broadcast_to(v, (16,))

    def row(h):
        return jnp.full((16,), h, i32)

    def do_graph(gi, carry):
        g = wid * GPW + gi
        pltpu.sync_copy(xp_hbm.at[pl.ds(g * NPG, NPG)], xp_v)
        pltpu.sync_copy(sc_hbm.at[g], sc_v)
        pltpu.sync_copy(eidx_hbm.at[g], idx_v)
        pltpu.sync_copy(ef_hbm.at[g], ef_v)

        for ch in range(4):
            off = ch * 16
            sv = idx_v[pl.ds(off, 16)]
            dv = idx_v[pl.ds(64 + off, 16)]
            eav = ef_v[pl.ds(off, 16)]
            emask = (lane + off) < MAX_OBS
            for h in range(HEADS):
                a_s = plsc.load_gather(sc_v, [row(h), sv], mask=emask)
                a_d = plsc.load_gather(sc_v, [row(h), 64 + dv], mask=emask)
                c_h = c_v[pl.ds(h * 16, 16)]
                al = a_s + a_d + eav * c_h
                al = jnp.where(al > 0, al, 0.2 * al)
                e_v[pl.ds(h * 64 + off, 16)] = jnp.where(
                    emask, jnp.exp(al), 0.0)

        if add_loops:
            for ch in range(4):
                off = ch * 16
                la = ef_v[pl.ds(64 + off, 16)]
                for h in range(HEADS):
                    s_s = plsc.load_gather(sc_v, [row(h), off + lane])
                    s_d = plsc.load_gather(sc_v, [row(h), 64 + off + lane])
                    c_h = c_v[pl.ds(h * 16, 16)]
                    al = s_s + s_d + la * c_h
                    al = jnp.where(al > 0, al, 0.2 * al)
                    el = jnp.exp(al)
                    el_v[pl.ds(h * 64 + off, 16)] = el
                    den_v[pl.ds(h * 64 + off, 16)] = el
        else:
            for i in range(32):
                den_v[pl.ds(i * 16, 16)] = jnp.zeros((16,), f32)

        def den_edge(e, c2):
            dvb = plsc.load_gather(idx_v, [bcast(64 + e)])
            e8 = plsc.load_gather(e_v, [lane * 64 + e], mask=mask8)
            plsc.addupdate_scatter(den_v, [lane * 64 + dvb], e8, mask=mask8)
            return c2
        lax.fori_loop(0, MAX_OBS, den_edge, 0)

        for i in range(32):
            sl = pl.ds(i * 16, 16)
            den_v[sl] = 1.0 / (den_v[sl] + 1e-16)

        if add_loops:
            def agg_node(n, c2):
                nb = bcast(n)
                el8 = plsc.load_gather(el_v, [lane * 64 + n], mask=mask8)
                r8 = plsc.load_gather(den_v, [lane * 64 + nb], mask=mask8)
                w8 = el8 * r8
                for h in range(HEADS):
                    wh = jnp.broadcast_to(w8[h], (16,))
                    xr = plsc.load_gather(xp_v, [nb, h * 16 + lane])
                    plsc.store_scatter(out_v, [nb, h * 16 + lane], wh * xr)
                return c2
            lax.fori_loop(0, NPG, agg_node, 0)
        else:
            def zero_node(n, c2):
                nb = bcast(n)
                zz = jnp.zeros((16,), f32)
                for h in range(HEADS):
                    plsc.store_scatter(out_v, [nb, h * 16 + lane], zz)
                return c2
            lax.fori_loop(0, NPG, zero_node, 0)

        def agg_edge(e, c2):
            svb = plsc.load_gather(idx_v, [bcast(e)])
            dvb = plsc.load_gather(idx_v, [bcast(64 + e)])
            e8 = plsc.load_gather(e_v, [lane * 64 + e], mask=mask8)
            r8 = plsc.load_gather(den_v, [lane * 64 + dvb], mask=mask8)
            w8 = e8 * r8
            for h in range(HEADS):
                wh = jnp.broadcast_to(w8[h], (16,))
                xr = plsc.load_gather(xp_v, [svb, h * 16 + lane])
                plsc.addupdate_scatter(out_v, [dvb, h * 16 + lane], wh * xr)
            return c2
        lax.fori_loop(0, MAX_OBS, agg_edge, 0)

        pltpu.sync_copy(out_v, out_hbm.at[pl.ds(g * NPG, NPG)])
        return carry

    lax.fori_loop(0, GPW, do_graph, 0)


def _make_sc(add_loops):
    mesh = plsc.VectorSubcoreMesh(core_axis_name="c", subcore_axis_name="s",
                                  num_cores=2, num_subcores=16)
    return pl.kernel(
        functools.partial(_sc_attn_body, add_loops),
        out_type=jax.ShapeDtypeStruct((N, HF), f32),
        mesh=mesh,
        compiler_params=pltpu.CompilerParams(use_tc_tiling_on_sc=False, needs_layout_passes=False),
        scratch_types=[
            pltpu.VMEM((NPG, HF), f32),
            pltpu.VMEM((NPG, HF), f32),
            pltpu.VMEM((HEADS, 2 * NPG), f32),
            pltpu.VMEM((128,), i32),
            pltpu.VMEM((128,), f32),
            pltpu.VMEM((1024,), f32),
            pltpu.VMEM((1024,), f32),
            pltpu.VMEM((1024,), f32),
            pltpu.VMEM((128,), f32),
            pltpu.VMEM((16,), f32),
        ],
    )


@functools.lru_cache(maxsize=None)
def _sc_kernel(add_loops):
    return _make_sc(add_loops)

_full = lambda shape: pl.BlockSpec(shape, lambda i: tuple(0 for _ in shape))


def _tc1(gfn, srcf, dstf, ea, W1, As, Ad):
    return pl.pallas_call(
        _tc1_body,
        grid=(B // GBT,),
        in_specs=[
            pl.BlockSpec((GBT, NNF, NPG), lambda i: (i, 0, 0)),
            pl.BlockSpec((GBT, MAX_OBS), lambda i: (i, 0)),
            pl.BlockSpec((GBT, MAX_OBS), lambda i: (i, 0)),
            pl.BlockSpec((GBT, MAX_OBS), lambda i: (i, 0)),
            _full((NNF, HF)), _full((HF, HEADS)), _full((HF, HEADS)),
        ],
        out_specs=[
            pl.BlockSpec((GBT * NPG, HF), lambda i: (i, 0)),
            pl.BlockSpec((GBT, HEADS, 2 * NPG), lambda i: (i, 0, 0)),
            pl.BlockSpec((GBT, 128), lambda i: (i, 0)),
            pl.BlockSpec((GBT, 128), lambda i: (i, 0)),
        ],
        out_shape=[
            jax.ShapeDtypeStruct((N, HF), f32),
            jax.ShapeDtypeStruct((B, HEADS, 2 * NPG), f32),
            jax.ShapeDtypeStruct((B, 128), i32),
            jax.ShapeDtypeStruct((B, 128), f32),
        ],
    )(gfn, srcf, dstf, ea, W1, As, Ad)


def _tc2(agg, b, W, As, Ad):
    return pl.pallas_call(
        _tc2_body,
        grid=(B // GBT,),
        in_specs=[
            pl.BlockSpec((GBT * NPG, HF), lambda i: (i, 0)),
            _full((1, HF)), _full((HF, HF)),
            _full((HF, HEADS)), _full((HF, HEADS)),
        ],
        out_specs=[
            pl.BlockSpec((GBT * NPG, HF), lambda i: (i, 0)),
            pl.BlockSpec((GBT, HEADS, 2 * NPG), lambda i: (i, 0, 0)),
        ],
        out_shape=[
            jax.ShapeDtypeStruct((N, HF), f32),
            jax.ShapeDtypeStruct((B, HEADS, 2 * NPG), f32),
        ],
    )(agg, b, W, As, Ad)


def _tc4(agg, b, fc1w, fc1b, fc2w, fc2b):
    return pl.pallas_call(
        _tc4_body,
        grid=(B // GBT,),
        in_specs=[
            pl.BlockSpec((GBT * NPG, HF), lambda i: (i, 0)),
            _full((1, HF)),
            _full((2 * HF, 4 * HID)), _full((1, 4 * HID)),
            _full((4 * HID, OUT_CH)), _full((1, OUT_CH)),
        ],
        out_specs=pl.BlockSpec((GBT * NUM_AGENTS, OUT_CH), lambda i: (i, 0)),
        out_shape=jax.ShapeDtypeStruct((B * NUM_AGENTS, OUT_CH), f32),
    )(agg, b, fc1w, fc1b, fc2w, fc2b)


@jax.jit
def kernel(tensor, W1, att_src1, att_dst1, W_edge1, att_edge1, b1,
           W2, att_src2, att_dst2, W_edge2, att_edge2, b2,
           W3, att_src3, att_dst3, W_edge3, att_edge3, b3,
           fc1_w, fc1_b, fc2_w, fc2_b):
    Bsz = tensor.shape[0]
    gf = tensor[:, 0, :]
    gfn = gf[:, :NFL].reshape(Bsz, NPG, NNF).transpose(0, 2, 1)
    srcf = gf[:, NFL:NFL + MAX_OBS]
    dstf = gf[:, NFL + MAX_OBS:NFL + 2 * MAX_OBS]
    ea = gf[:, NFL + 2 * MAX_OBS:NFL + 3 * MAX_OBS]

    eye = jnp.eye(HEADS, dtype=f32)

    def prep(a_srd, a_dst, We, a_e):
        As = (a_srd[0][:, :, None] * eye[:, None, :]).reshape(HF, HEADS)
        Ad = (a_dst[0][:, :, None] * eye[:, None, :]).reshape(HF, HEADS)
        c = (We[0].reshape(HEADS, HID) * a_e[0]).sum(-1)
        return As, Ad, jnp.repeat(c, HID)

    As1, Ad1, c1 = prep(att_src1, att_dst1, W_edge1, att_edge1)
    As2, Ad2, c2 = prep(att_src2, att_dst2, W_edge2, att_edge2)
    As3, Ad3, c3 = prep(att_src3, att_dst3, W_edge3, att_edge3)

    xp1, sc1, eidx, efeat = _tc1(gfn, srcf, dstf, ea, W1, As1, Ad1)
    agg1 = _sc_kernel(False)(xp1, sc1, eidx, efeat, c1)
    xp2, sc2 = _tc2(agg1, b1.reshape(1, HF), W2, As2, Ad2)
    agg2 = _sc_kernel(True)(xp2, sc2, eidx, efeat, c2)
    xp3, sc3 = _tc2(agg2, b2.reshape(1, HF), W3, As3, Ad3)
    agg3 = _sc_kernel(True)(xp3, sc3, eidx, efeat, c3)
    pred = _tc4(agg3, b3.reshape(1, HF), fc1_w, fc1_b.reshape(1, 4 * HID),
                fc2_w, fc2_b.reshape(1, OUT_CH))
    return pred.reshape(Bsz, NUM_AGENTS, OUT_CH)
